# baseline (device time: 12074 ns/iter reference)
import jax
import jax.numpy as jnp
from jax import lax
from jax.experimental import pallas as pl
from jax.experimental.pallas import tpu as pltpu

NCHUNK = 4


def kernel(x):
    _, m, n2 = x.shape
    n = n2 // 2
    rows = m // NCHUNK

    def body(x_hbm, out_hbm, x_vmem, send_buf, comm_ref, out_vmem,
             in_sems, out_sems, send_sems, recv_sems):
        px = lax.axis_index("x")
        py = lax.axis_index("y")
        pz = lax.axis_index("z")
        peer = 1 - px

        copies_in = []
        for c in range(NCHUNK):
            r = pl.ds(c * rows, rows)
            cp = pltpu.make_async_copy(x_hbm.at[0, r, :], x_vmem.at[r, :],
                                       in_sems.at[c])
            cp.start()
            copies_in.append(cp)

        barrier_sem = pltpu.get_barrier_semaphore()
        pl.semaphore_signal(
            barrier_sem, inc=1,
            device_id=(peer, py, pz), device_id_type=pl.DeviceIdType.MESH,
        )
        pl.semaphore_wait(barrier_sem, 1)

        rdmas = []
        for c in range(NCHUNK):
            r = pl.ds(c * rows, rows)
            copies_in[c].wait()
            send_buf[r, :] = x_vmem[r, pl.ds(peer * n, n)].astype(jnp.bfloat16)
            rdma = pltpu.make_async_remote_copy(
                src_ref=send_buf.at[r, :],
                dst_ref=comm_ref.at[r, :],
                send_sem=send_sems.at[c],
                recv_sem=recv_sems.at[c],
                device_id=(peer, py, pz),
                device_id_type=pl.DeviceIdType.MESH,
            )
            rdma.start()
            rdmas.append(rdma)

        copies_out = []
        for c in range(NCHUNK):
            r = pl.ds(c * rows, rows)
            rdmas[c].wait_recv()
            out_vmem[r, :] = (
                x_vmem[r, pl.ds(px * n, n)]
                + comm_ref[r, :].astype(jnp.float32)
            )
            cp = pltpu.make_async_copy(out_vmem.at[r, :], out_hbm.at[r, :],
                                       out_sems.at[c])
            cp.start()
            copies_out.append(cp)

        for c in range(NCHUNK):
            copies_out[c].wait()
            rdmas[c].wait_send()

    return pl.pallas_call(
        body,
        out_shape=jax.ShapeDtypeStruct((m, n), jnp.float32),
        in_specs=[pl.BlockSpec(memory_space=pl.ANY)],
        out_specs=pl.BlockSpec(memory_space=pl.ANY),
        scratch_shapes=[
            pltpu.VMEM((m, n2), jnp.float32),
            pltpu.VMEM((m, n), jnp.bfloat16),
            pltpu.VMEM((m, n), jnp.bfloat16),
            pltpu.VMEM((m, n), jnp.float32),
            pltpu.SemaphoreType.DMA((NCHUNK,)),
            pltpu.SemaphoreType.DMA((NCHUNK,)),
            pltpu.SemaphoreType.DMA((NCHUNK,)),
            pltpu.SemaphoreType.DMA((NCHUNK,)),
        ],
        compiler_params=pltpu.CompilerParams(collective_id=0),
    )(x)


# device time: 11767 ns/iter; 1.0261x vs baseline; 1.0261x over previous
import jax
import jax.numpy as jnp
from jax import lax
from jax.experimental import pallas as pl
from jax.experimental.pallas import tpu as pltpu

NCHUNK = 4


def kernel(x):
    _, m, n2 = x.shape
    n = n2 // 2
    rows = m // NCHUNK

    def body(x_ref, out_ref, send_buf, comm_ref, send_sems, recv_sems):
        px = lax.axis_index("x")
        py = lax.axis_index("y")
        pz = lax.axis_index("z")
        peer = 1 - px

        barrier_sem = pltpu.get_barrier_semaphore()
        pl.semaphore_signal(
            barrier_sem, inc=1,
            device_id=(peer, py, pz), device_id_type=pl.DeviceIdType.MESH,
        )

        r0 = pl.ds(0, rows)
        send_buf[r0, :] = x_ref[0, r0, pl.ds(peer * n, n)].astype(jnp.bfloat16)

        pl.semaphore_wait(barrier_sem, 1)

        rdmas = []
        for c in range(NCHUNK):
            r = pl.ds(c * rows, rows)
            if c > 0:
                send_buf[r, :] = x_ref[0, r, pl.ds(peer * n, n)].astype(
                    jnp.bfloat16)
            rdma = pltpu.make_async_remote_copy(
                src_ref=send_buf.at[r, :],
                dst_ref=comm_ref.at[r, :],
                send_sem=send_sems.at[c],
                recv_sem=recv_sems.at[c],
                device_id=(peer, py, pz),
                device_id_type=pl.DeviceIdType.MESH,
            )
            rdma.start()
            rdmas.append(rdma)

        for c in range(NCHUNK):
            r = pl.ds(c * rows, rows)
            rdmas[c].wait_recv()
            out_ref[r, :] = (
                x_ref[0, r, pl.ds(px * n, n)]
                + comm_ref[r, :].astype(jnp.float32)
            )
        for c in range(NCHUNK):
            rdmas[c].wait_send()

    return pl.pallas_call(
        body,
        out_shape=jax.ShapeDtypeStruct((m, n), jnp.float32),
        in_specs=[pl.BlockSpec(memory_space=pltpu.VMEM)],
        out_specs=pl.BlockSpec(memory_space=pltpu.VMEM),
        scratch_shapes=[
            pltpu.VMEM((m, n), jnp.bfloat16),
            pltpu.VMEM((m, n), jnp.bfloat16),
            pltpu.SemaphoreType.DMA((NCHUNK,)),
            pltpu.SemaphoreType.DMA((NCHUNK,)),
        ],
        compiler_params=pltpu.CompilerParams(collective_id=0),
    )(x)
